# Initial kernel scaffold; baseline (speedup 1.0000x reference)
import jax
import jax.numpy as jnp
from jax.experimental import pallas as pl


def kernel(bboxes_txtytwth, anchors, scores, image_shape):
    raise NotImplementedError("write your pallas kernel here")


# speedup vs baseline: 21.2530x; 21.2530x over previous
import functools

import jax
import jax.numpy as jnp
from jax import lax
from jax.experimental import pallas as pl
from jax.experimental.pallas import tpu as pltpu

_N = 20000
_ROWS = 160
_LANES = 128
_NP = _ROWS * _LANES
_K = 300
_IOU_THR = 0.7


def _nms_body(d0, d1, d2, d3, a0, a1, a2, a3, s_in, img,
              out_ref, y1s, x1s, y2s, x2s, areas, live):
    h = img[0, 0]
    w = img[0, 1]

    heights = a2[...] - a0[...]
    widths = a3[...] - a1[...]
    ctr_y = a0[...] + 0.5 * heights
    ctr_x = a1[...] + 0.5 * widths
    pred_cy = d0[...] * heights + ctr_y
    pred_cx = d1[...] * widths + ctr_x
    pred_h = jnp.exp(d2[...]) * heights
    pred_w = jnp.exp(d3[...]) * widths
    y1 = jnp.minimum(jnp.maximum(pred_cy - 0.5 * pred_h, 0.0), h)
    x1 = jnp.minimum(jnp.maximum(pred_cx - 0.5 * pred_w, 0.0), w)
    y2 = jnp.minimum(jnp.maximum(pred_cy + 0.5 * pred_h, 0.0), h)
    x2 = jnp.minimum(jnp.maximum(pred_cx + 0.5 * pred_w, 0.0), w)
    y1s[...] = y1
    x1s[...] = x1
    y2s[...] = y2
    x2s[...] = x2
    areas[...] = (y2 - y1) * (x2 - x1)
    live[...] = s_in[...]

    iota = (lax.broadcasted_iota(jnp.int32, (_ROWS, _LANES), 0) * _LANES
            + lax.broadcasted_iota(jnp.int32, (_ROWS, _LANES), 1))
    lane = lax.broadcasted_iota(jnp.int32, (1, _LANES), 1)
    neg_inf = jnp.float32(-jnp.inf)

    def body(i, _):
        s = live[...]
        m = jnp.max(s)
        idx = jnp.min(jnp.where(s == m, iota, _NP))
        onehot = iota == idx
        cy1 = y1s[...]
        cx1 = x1s[...]
        cy2 = y2s[...]
        cx2 = x2s[...]
        by1 = jnp.sum(jnp.where(onehot, cy1, 0.0))
        bx1 = jnp.sum(jnp.where(onehot, cx1, 0.0))
        by2 = jnp.sum(jnp.where(onehot, cy2, 0.0))
        bx2 = jnp.sum(jnp.where(onehot, cx2, 0.0))
        yy1 = jnp.maximum(by1, cy1)
        xx1 = jnp.maximum(bx1, cx1)
        yy2 = jnp.minimum(by2, cy2)
        xx2 = jnp.minimum(bx2, cx2)
        inter = jnp.maximum(yy2 - yy1, 0.0) * jnp.maximum(xx2 - xx1, 0.0)
        area_b = (by2 - by1) * (bx2 - bx1)
        union = jnp.maximum(area_b + areas[...] - inter, 1e-8)
        iou = inter / union
        suppress = (iou > _IOU_THR) | onehot
        live[...] = jnp.where(suppress, neg_inf, s)
        valid = m > neg_inf
        row = jnp.where(valid & (lane == 0), by1, 0.0)
        row = jnp.where(valid & (lane == 1), bx1, row)
        row = jnp.where(valid & (lane == 2), by2, row)
        row = jnp.where(valid & (lane == 3), bx2, row)
        out_ref[pl.ds(i, 1), :] = row
        return 0

    lax.fori_loop(0, _K, body, 0)


@jax.jit
def kernel(bboxes_txtytwth, anchors, scores, image_shape):
    pad = _NP - _N

    def prep(col):
        return jnp.pad(col, (0, pad)).reshape(_ROWS, _LANES)

    args = [prep(bboxes_txtytwth[:, c]) for c in range(4)]
    args += [prep(anchors[:, c]) for c in range(4)]
    args.append(jnp.pad(scores, (0, pad), constant_values=-jnp.inf)
                .reshape(_ROWS, _LANES))
    args.append(image_shape.reshape(1, 2))

    out = pl.pallas_call(
        _nms_body,
        out_shape=jax.ShapeDtypeStruct((_K + 4, _LANES), jnp.float32),
        scratch_shapes=[pltpu.VMEM((_ROWS, _LANES), jnp.float32)] * 6,
    )(*args)
    return out[:_K, :4]
